# baseline (speedup 1.0000x reference)
import functools

import jax
import jax.numpy as jnp
from jax import lax
from jax.experimental import pallas as pl
from jax.experimental.pallas import tpu as pltpu
from jax.experimental.pallas import tpu_sc as plsc

N = 4096
NT = 16384
C = 128
NCH = N // C
RB = 512
NRB = N // RB
NW = 32
QPW = NT // NW
CLIP_LO = -2.0
CLIP_HI = 2.0
_F32 = jnp.float32
_I32 = jnp.int32


def _bitonic_pairs(x, ypay, p):
    for k in range(12):
        asc = ((p >> (k + 1)) & 1) == 0
        for j in range(k, -1, -1):
            if j < 7:
                ax, sz, s = 1, C, 1 << j
            else:
                ax, sz, s = 0, NCH, 1 << (j - 7)
            is_low = ((p >> j) & 1) == 0
            pvx = jnp.where(is_low, pltpu.roll(x, sz - s, axis=ax),
                            pltpu.roll(x, s, axis=ax))
            pvy = jnp.where(is_low, pltpu.roll(ypay, sz - s, axis=ax),
                            pltpu.roll(ypay, s, axis=ax))
            want_min = is_low == asc
            take = (want_min & (pvx < x)) | (~want_min & (pvx > x))
            x = jnp.where(take, pvx, x)
            ypay = jnp.where(take, pvy, ypay)
    return x, ypay


def _fit_body(x2d_ref, y2d_ref,
              ux_ref, uyc_ref, nu_ref, xmin_ref, xmax_ref,
              sx_ref, sy_ref, seg_ref):
    lane = lax.broadcasted_iota(_I32, (1, C), 1)
    sub128 = lax.broadcasted_iota(_I32, (C, 1), 0)
    sub512 = lax.broadcasted_iota(_I32, (RB, 1), 0)
    sub32 = lax.broadcasted_iota(_I32, (NCH, 1), 0)
    lanei = lax.broadcasted_iota(_I32, (NCH, C), 1)
    subi = lax.broadcasted_iota(_I32, (NCH, C), 0)
    eye = sub128 == lane
    NEG = _F32(-jnp.inf)
    POS = _F32(jnp.inf)

    sx2d, sy2d = _bitonic_pairs(x2d_ref[...], y2d_ref[...],
                                subi * C + lanei)
    sx_ref[...] = sx2d
    sy_ref[...] = sy2d

    def prev_elem(v, fill):
        r1 = pltpu.roll(v, 1, axis=1)
        pv = jnp.where(lanei == 0, pltpu.roll(r1, 1, axis=0), r1)
        return jnp.where(subi * C + lanei == 0, fill, pv)

    def cumsum2d(v):
        s = 1
        while s < C:
            sh = pltpu.roll(v, s, axis=1)
            v = v + jnp.where(lanei >= s, sh, jnp.zeros((), v.dtype))
            s *= 2
        rt = v[:, C - 1:C]
        t = rt
        s = 1
        while s < NCH:
            sh = pltpu.roll(t, s, axis=0)
            t = t + jnp.where(sub32 >= s, sh, jnp.zeros((), v.dtype))
            s *= 2
        return v + (t - rt)

    flag = (subi * C + lanei == 0) | (sx2d != prev_elem(sx2d, NEG))
    flag_i = flag.astype(_I32)
    seg_ref[...] = cumsum2d(flag_i) - 1
    n_u = jnp.sum(jnp.sum(flag_i, axis=1, keepdims=True),
                  axis=0, keepdims=True)
    n_u_s = n_u[0, 0]
    band = (_I32(RB - 1) + (_I32(N) - n_u_s)) // _I32(C) + 1

    def s4(rb, _):
        ug = rb * RB + sub512
        c0 = rb * (RB // C)

        def inner(t, carry):
            cnt, sm, mx = carry
            c = c0 + t
            oh = seg_ref[pl.ds(c, 1), :] == ug
            syc = sy_ref[pl.ds(c, 1), :]
            sxc = sx_ref[pl.ds(c, 1), :]
            cnt = cnt + jnp.sum(jnp.where(oh, _F32(1.0), _F32(0.0)),
                                axis=1, keepdims=True)
            sm = sm + jnp.sum(jnp.where(oh, syc, _F32(0.0)),
                              axis=1, keepdims=True)
            mx = jnp.maximum(mx, jnp.max(jnp.where(oh, sxc, NEG),
                                         axis=1, keepdims=True))
            return cnt, sm, mx

        ntr = jnp.minimum(band, _I32(NCH) - c0)
        cnt, sm, mx = lax.fori_loop(
            0, ntr, inner,
            (jnp.zeros((RB, 1), _F32), jnp.zeros((RB, 1), _F32),
             jnp.full((RB, 1), NEG)))
        uyc_ref[pl.ds(rb * RB, RB), :] = sm / jnp.maximum(cnt, _F32(1.0))
        ux_ref[pl.ds(rb * RB, RB), :] = jnp.where(ug >= n_u, POS, mx)
        return 0
    lax.fori_loop(0, NRB, s4, 0, unroll=False)

    nu_ref[...] = n_u
    x2d = x2d_ref[...]
    xmin_ref[...] = jnp.min(jnp.min(x2d, axis=1, keepdims=True),
                            axis=0, keepdims=True)
    xmax_ref[...] = jnp.max(jnp.max(x2d, axis=1, keepdims=True),
                            axis=0, keepdims=True)


_FIT_OUT = [
    jax.ShapeDtypeStruct((N, 1), _F32),
    jax.ShapeDtypeStruct((N, 1), _F32),
    jax.ShapeDtypeStruct((1, 1), _I32),
    jax.ShapeDtypeStruct((1, 1), _F32),
    jax.ShapeDtypeStruct((1, 1), _F32),
]

_FIT_SCRATCH = [
    pltpu.VMEM((NCH, C), _F32),
    pltpu.VMEM((NCH, C), _F32),
    pltpu.VMEM((NCH, C), _I32),
]

_fit = pl.pallas_call(
    _fit_body,
    out_shape=_FIT_OUT,
    scratch_shapes=_FIT_SCRATCH,
)


def _pava_body(uy_ref, nu_ref, pv_ref, ps_ref, np_ref, sv_ref, sw_ref):
    def _s(v):
        return v if getattr(v, "ndim", 0) == 0 else jnp.reshape(v, ())
    n_u = _s(nu_ref[0])

    def o_cond(c):
        i, top, tv, tw, tv2, tw2 = c
        return i < n_u

    def o_body(c):
        i, top, tv, tw, tv2, tw2 = c

        def m_cond(mc):
            v, w, t, mtv, mtw, mtv2, mtw2 = mc
            return (t >= 0) & (mtv >= v)

        def m_body(mc):
            v, w, t, mtv, mtw, mtv2, mtw2 = mc
            nw = mtw + w
            v = (mtv * mtw + v * w) / nw
            t = t - 1
            g = jnp.maximum(t - 1, 0)
            return v, nw, t, mtv2, mtw2, _s(sv_ref[g]), _s(sw_ref[g])

        v, w, top, tv, tw, tv2, tw2 = lax.while_loop(
            m_cond, m_body, (_s(uy_ref[i]), _F32(1.0), top, tv, tw, tv2, tw2))
        top = top + 1
        sv_ref[top] = v
        sw_ref[top] = w
        pv_ref[top] = jnp.clip(v, _F32(CLIP_LO), _F32(CLIP_HI))
        ps_ref[top] = i + 1 - w.astype(_I32)
        return i + 1, top, v, w, tv, tw

    _, top, _, _, _, _ = lax.while_loop(
        o_cond, o_body, (_I32(0), _I32(-1), _F32(0.0), _F32(0.0),
                         _F32(0.0), _F32(0.0)))
    np_ref[0] = top + 1


_PAVA_SPECS = dict(
    in_specs=[pl.BlockSpec(memory_space=pltpu.SMEM),
              pl.BlockSpec(memory_space=pltpu.SMEM)],
    out_specs=[pl.BlockSpec(memory_space=pltpu.SMEM),
               pl.BlockSpec(memory_space=pltpu.SMEM),
               pl.BlockSpec(memory_space=pltpu.SMEM)],
    out_shape=[jax.ShapeDtypeStruct((N,), _F32),
               jax.ShapeDtypeStruct((N,), _I32),
               jax.ShapeDtypeStruct((1,), _I32)],
    scratch_shapes=[pltpu.SMEM((N,), _F32),
                    pltpu.SMEM((N,), _F32)],
)

_pava = pl.pallas_call(_pava_body, **_PAVA_SPECS)


def _predict(T, ux1, pv1, ps1, xmin16, xmax16, nhi16, npl16):
    mesh = plsc.VectorSubcoreMesh(core_axis_name="c", subcore_axis_name="s")

    @functools.partial(
        pl.kernel, mesh=mesh,
        out_type=jax.ShapeDtypeStruct((NT,), _F32),
        compiler_params=pltpu.CompilerParams(needs_layout_passes=False),
        scratch_types=[
            pltpu.VMEM((N,), _F32),
            pltpu.VMEM((N,), _F32),
            pltpu.VMEM((N,), _I32),
            pltpu.VMEM((QPW,), _F32),
            pltpu.VMEM((QPW,), _F32),
            pltpu.VMEM((16,), _F32),
            pltpu.VMEM((16,), _F32),
            pltpu.VMEM((16,), _I32),
            pltpu.VMEM((16,), _I32),
        ],
    )
    def k(t_hbm, ux_hbm, pv_hbm, ps_hbm, xmin_hbm, xmax_hbm, nhi_hbm,
          npl_hbm, out_hbm,
          ux_v, pv_v, ps_v, t_v, o_v, xmin_v, xmax_v, nhi_v, npl_v):
        wid = lax.axis_index("s") * 2 + lax.axis_index("c")
        base = wid * QPW
        pltpu.sync_copy(ux_hbm, ux_v)
        pltpu.sync_copy(pv_hbm, pv_v)
        pltpu.sync_copy(ps_hbm, ps_v)
        pltpu.sync_copy(t_hbm.at[pl.ds(base, QPW)], t_v)
        pltpu.sync_copy(xmin_hbm, xmin_v)
        pltpu.sync_copy(xmax_hbm, xmax_v)
        pltpu.sync_copy(nhi_hbm, nhi_v)
        pltpu.sync_copy(npl_hbm, npl_v)
        xmin = xmin_v[...]
        xmax = xmax_v[...]
        nhi = nhi_v[...]
        npl = npl_v[...]

        def body(g, acc):
            t = t_v[pl.ds(g * 16, 16)]
            tc = jnp.minimum(jnp.maximum(t, xmin), xmax)
            pos = jnp.zeros((16,), _I32)
            s = N // 2
            while s >= 1:
                cand = pos + s
                probe = plsc.load_gather(ux_v, [cand - 1])
                pos = jnp.where(probe <= tc, cand, pos)
                s //= 2
            idx = jnp.clip(pos - 1, 0, nhi)
            xb = plsc.load_gather(ux_v, [idx])
            xa = plsc.load_gather(ux_v, [idx + 1])
            pp = jnp.zeros((16,), _I32)
            s = N // 2
            while s >= 1:
                cand = pp + s
                probe = plsc.load_gather(ps_v, [cand - 1])
                take = (cand <= npl) & (probe <= idx)
                pp = jnp.where(take, cand, pp)
                s //= 2
            pb = pp - 1
            yb = plsc.load_gather(pv_v, [pb])
            pb1 = jnp.minimum(pb + 1, npl - 1)
            nxt = plsc.load_gather(ps_v, [pb1])
            same = (pb1 == pb) | (idx + 1 < nxt)
            ya = jnp.where(same, yb, plsc.load_gather(pv_v, [pb1]))
            slope = (ya - yb) / (xa - xb)
            o_v[pl.ds(g * 16, 16)] = yb + slope * (tc - xb)
            return acc

        lax.fori_loop(0, QPW // 16, body, 0)
        pltpu.sync_copy(o_v, out_hbm.at[pl.ds(base, QPW)])

    return k(T, ux1, pv1, ps1, xmin16, xmax16, nhi16, npl16)


def kernel(X, y, T):
    ux, uy, nu, xmin, xmax = _fit(X.reshape(NCH, C), y.reshape(NCH, C))
    pv, ps, npl = _pava(uy.reshape(N), nu.reshape(1))
    nhi = jnp.maximum(nu[0, 0] - 2, 0).astype(_I32)
    return _predict(
        T, ux.reshape(N), pv, ps,
        jnp.full((16,), xmin[0, 0], _F32),
        jnp.full((16,), xmax[0, 0], _F32),
        jnp.full((16,), nhi, _I32),
        jnp.full((16,), npl[0], _I32),
    )


# speedup vs baseline: 1.0450x; 1.0450x over previous
import functools

import jax
import jax.numpy as jnp
from jax import lax
from jax.experimental import pallas as pl
from jax.experimental.pallas import tpu as pltpu
from jax.experimental.pallas import tpu_sc as plsc

N = 4096
NT = 16384
C = 128
NCH = N // C
RB = 512
NRB = N // RB
NW = 32
QPW = NT // NW
CLIP_LO = -2.0
CLIP_HI = 2.0
_F32 = jnp.float32
_I32 = jnp.int32


def _bitonic_pairs(x, ypay, p):
    for k in range(12):
        asc = ((p >> (k + 1)) & 1) == 0
        for j in range(k, -1, -1):
            if j < 7:
                ax, sz, s = 1, C, 1 << j
            else:
                ax, sz, s = 0, NCH, 1 << (j - 7)
            is_low = ((p >> j) & 1) == 0
            pvx = jnp.where(is_low, pltpu.roll(x, sz - s, axis=ax),
                            pltpu.roll(x, s, axis=ax))
            pvy = jnp.where(is_low, pltpu.roll(ypay, sz - s, axis=ax),
                            pltpu.roll(ypay, s, axis=ax))
            want_min = is_low == asc
            take = (want_min & (pvx < x)) | (~want_min & (pvx > x))
            x = jnp.where(take, pvx, x)
            ypay = jnp.where(take, pvy, ypay)
    return x, ypay


def _fit_body(x2d_ref, y2d_ref,
              ux_ref, uyc_ref, nu_ref, xmin_ref, xmax_ref,
              sx_ref, sy_ref, seg_ref):
    lane = lax.broadcasted_iota(_I32, (1, C), 1)
    sub128 = lax.broadcasted_iota(_I32, (C, 1), 0)
    sub512 = lax.broadcasted_iota(_I32, (RB, 1), 0)
    sub32 = lax.broadcasted_iota(_I32, (NCH, 1), 0)
    lanei = lax.broadcasted_iota(_I32, (NCH, C), 1)
    subi = lax.broadcasted_iota(_I32, (NCH, C), 0)
    eye = sub128 == lane
    NEG = _F32(-jnp.inf)
    POS = _F32(jnp.inf)

    sx2d, sy2d = _bitonic_pairs(x2d_ref[...], y2d_ref[...],
                                subi * C + lanei)
    sx_ref[...] = sx2d
    sy_ref[...] = sy2d

    def prev_elem(v, fill):
        r1 = pltpu.roll(v, 1, axis=1)
        pv = jnp.where(lanei == 0, pltpu.roll(r1, 1, axis=0), r1)
        return jnp.where(subi * C + lanei == 0, fill, pv)

    def cumsum2d(v):
        s = 1
        while s < C:
            sh = pltpu.roll(v, s, axis=1)
            v = v + jnp.where(lanei >= s, sh, jnp.zeros((), v.dtype))
            s *= 2
        rt = v[:, C - 1:C]
        t = rt
        s = 1
        while s < NCH:
            sh = pltpu.roll(t, s, axis=0)
            t = t + jnp.where(sub32 >= s, sh, jnp.zeros((), v.dtype))
            s *= 2
        return v + (t - rt)

    flag = (subi * C + lanei == 0) | (sx2d != prev_elem(sx2d, NEG))
    flag_i = flag.astype(_I32)
    seg_ref[...] = cumsum2d(flag_i) - 1
    n_u = jnp.sum(jnp.sum(flag_i, axis=1, keepdims=True),
                  axis=0, keepdims=True)
    n_u_s = n_u[0, 0]
    band = (_I32(RB - 1) + (_I32(N) - n_u_s)) // _I32(C) + 1

    def s4(rb, _):
        ug = rb * RB + sub512
        c0 = rb * (RB // C)

        def inner(t, carry):
            cnt, sm, mx = carry
            c = c0 + t
            oh = seg_ref[pl.ds(c, 1), :] == ug
            syc = sy_ref[pl.ds(c, 1), :]
            sxc = sx_ref[pl.ds(c, 1), :]
            cnt = cnt + jnp.sum(jnp.where(oh, _F32(1.0), _F32(0.0)),
                                axis=1, keepdims=True)
            sm = sm + jnp.sum(jnp.where(oh, syc, _F32(0.0)),
                              axis=1, keepdims=True)
            mx = jnp.maximum(mx, jnp.max(jnp.where(oh, sxc, NEG),
                                         axis=1, keepdims=True))
            return cnt, sm, mx

        ntr = jnp.minimum(band, _I32(NCH) - c0)
        cnt, sm, mx = lax.fori_loop(
            0, ntr, inner,
            (jnp.zeros((RB, 1), _F32), jnp.zeros((RB, 1), _F32),
             jnp.full((RB, 1), NEG)))
        uyc_ref[pl.ds(rb * RB, RB), :] = sm / jnp.maximum(cnt, _F32(1.0))
        ux_ref[pl.ds(rb * RB, RB), :] = jnp.where(ug >= n_u, POS, mx)
        return 0
    lax.fori_loop(0, NRB, s4, 0, unroll=False)

    nu_ref[...] = n_u
    x2d = x2d_ref[...]
    xmin_ref[...] = jnp.min(jnp.min(x2d, axis=1, keepdims=True),
                            axis=0, keepdims=True)
    xmax_ref[...] = jnp.max(jnp.max(x2d, axis=1, keepdims=True),
                            axis=0, keepdims=True)


_FIT_OUT = [
    jax.ShapeDtypeStruct((N, 1), _F32),
    jax.ShapeDtypeStruct((N, 1), _F32),
    jax.ShapeDtypeStruct((1, 1), _I32),
    jax.ShapeDtypeStruct((1, 1), _F32),
    jax.ShapeDtypeStruct((1, 1), _F32),
]

_FIT_SCRATCH = [
    pltpu.VMEM((NCH, C), _F32),
    pltpu.VMEM((NCH, C), _F32),
    pltpu.VMEM((NCH, C), _I32),
]

_fit = pl.pallas_call(
    _fit_body,
    out_shape=_FIT_OUT,
    scratch_shapes=_FIT_SCRATCH,
)


def _pava_body(uy_ref, nu_ref, pv_ref, ps_ref, np_ref, sw_ref):
    def _s(v):
        return v if getattr(v, "ndim", 0) == 0 else jnp.reshape(v, ())
    n_u = _s(nu_ref[0])

    def o_cond(c):
        i, top, tv, tw = c
        return i < n_u

    def o_body(c):
        i, top, tv, tw = c

        def m_cond(mc):
            v, w, t, mtv, mtw = mc
            return (t >= 0) & (mtv >= v)

        def m_body(mc):
            v, w, t, mtv, mtw = mc
            nw = mtw + w
            v = (mtv * mtw + v * w) / nw
            t = t - 1
            g = jnp.maximum(t, 0)
            return v, nw, t, _s(pv_ref[g]), _s(sw_ref[g])

        v, w, top, tv, tw = lax.while_loop(
            m_cond, m_body, (_s(uy_ref[i]), _F32(1.0), top, tv, tw))
        top = top + 1
        pv_ref[top] = v
        sw_ref[top] = w
        ps_ref[top] = i + 1 - w.astype(_I32)
        return i + 1, top, v, w

    _, top, _, _ = lax.while_loop(
        o_cond, o_body, (_I32(0), _I32(-1), _F32(0.0), _F32(0.0)))
    np_ref[0] = top + 1


_PAVA_SPECS = dict(
    in_specs=[pl.BlockSpec(memory_space=pltpu.SMEM),
              pl.BlockSpec(memory_space=pltpu.SMEM)],
    out_specs=[pl.BlockSpec(memory_space=pltpu.SMEM),
               pl.BlockSpec(memory_space=pltpu.SMEM),
               pl.BlockSpec(memory_space=pltpu.SMEM)],
    out_shape=[jax.ShapeDtypeStruct((N,), _F32),
               jax.ShapeDtypeStruct((N,), _I32),
               jax.ShapeDtypeStruct((1,), _I32)],
    scratch_shapes=[pltpu.SMEM((N,), _F32)],
)

_pava = pl.pallas_call(_pava_body, **_PAVA_SPECS)


def _predict(T, ux1, pv1, ps1, xmin16, xmax16, nhi16, npl16):
    mesh = plsc.VectorSubcoreMesh(core_axis_name="c", subcore_axis_name="s")

    @functools.partial(
        pl.kernel, mesh=mesh,
        out_type=jax.ShapeDtypeStruct((NT,), _F32),
        compiler_params=pltpu.CompilerParams(needs_layout_passes=False),
        scratch_types=[
            pltpu.VMEM((N,), _F32),
            pltpu.VMEM((N,), _F32),
            pltpu.VMEM((N,), _I32),
            pltpu.VMEM((QPW,), _F32),
            pltpu.VMEM((QPW,), _F32),
            pltpu.VMEM((16,), _F32),
            pltpu.VMEM((16,), _F32),
            pltpu.VMEM((16,), _I32),
            pltpu.VMEM((16,), _I32),
        ],
    )
    def k(t_hbm, ux_hbm, pv_hbm, ps_hbm, xmin_hbm, xmax_hbm, nhi_hbm,
          npl_hbm, out_hbm,
          ux_v, pv_v, ps_v, t_v, o_v, xmin_v, xmax_v, nhi_v, npl_v):
        wid = lax.axis_index("s") * 2 + lax.axis_index("c")
        base = wid * QPW
        pltpu.sync_copy(ux_hbm, ux_v)
        pltpu.sync_copy(pv_hbm, pv_v)
        pltpu.sync_copy(ps_hbm, ps_v)
        pltpu.sync_copy(t_hbm.at[pl.ds(base, QPW)], t_v)
        pltpu.sync_copy(xmin_hbm, xmin_v)
        pltpu.sync_copy(xmax_hbm, xmax_v)
        pltpu.sync_copy(nhi_hbm, nhi_v)
        pltpu.sync_copy(npl_hbm, npl_v)
        xmin = xmin_v[...]
        xmax = xmax_v[...]
        nhi = nhi_v[...]
        npl = npl_v[...]

        def body(g, acc):
            t = t_v[pl.ds(g * 16, 16)]
            tc = jnp.minimum(jnp.maximum(t, xmin), xmax)
            pos = jnp.zeros((16,), _I32)
            s = N // 2
            while s >= 1:
                cand = pos + s
                probe = plsc.load_gather(ux_v, [cand - 1])
                pos = jnp.where(probe <= tc, cand, pos)
                s //= 2
            idx = jnp.clip(pos - 1, 0, nhi)
            xb = plsc.load_gather(ux_v, [idx])
            xa = plsc.load_gather(ux_v, [idx + 1])
            pp = jnp.zeros((16,), _I32)
            s = N // 2
            while s >= 1:
                cand = pp + s
                probe = plsc.load_gather(ps_v, [cand - 1])
                take = (cand <= npl) & (probe <= idx)
                pp = jnp.where(take, cand, pp)
                s //= 2
            pb = pp - 1
            yb = jnp.clip(plsc.load_gather(pv_v, [pb]),
                          _F32(CLIP_LO), _F32(CLIP_HI))
            pb1 = jnp.minimum(pb + 1, npl - 1)
            nxt = plsc.load_gather(ps_v, [pb1])
            same = (pb1 == pb) | (idx + 1 < nxt)
            ya = jnp.where(same, yb,
                           jnp.clip(plsc.load_gather(pv_v, [pb1]),
                                    _F32(CLIP_LO), _F32(CLIP_HI)))
            slope = (ya - yb) / (xa - xb)
            o_v[pl.ds(g * 16, 16)] = yb + slope * (tc - xb)
            return acc

        lax.fori_loop(0, QPW // 16, body, 0)
        pltpu.sync_copy(o_v, out_hbm.at[pl.ds(base, QPW)])

    return k(T, ux1, pv1, ps1, xmin16, xmax16, nhi16, npl16)


def kernel(X, y, T):
    ux, uy, nu, xmin, xmax = _fit(X.reshape(NCH, C), y.reshape(NCH, C))
    pv, ps, npl = _pava(uy.reshape(N), nu.reshape(1))
    nhi = jnp.maximum(nu[0, 0] - 2, 0).astype(_I32)
    return _predict(
        T, ux.reshape(N), pv, ps,
        jnp.full((16,), xmin[0, 0], _F32),
        jnp.full((16,), xmax[0, 0], _F32),
        jnp.full((16,), nhi, _I32),
        jnp.full((16,), npl[0], _I32),
    )
